# baseline (speedup 1.0000x reference)
import functools

import jax
import jax.numpy as jnp
from jax import lax
from jax.experimental import pallas as pl
from jax.experimental.pallas import tpu as pltpu
from jax.experimental.pallas import tpu_sc as plsc

NC = 2
NS = 16
NW = NC * NS
LANES = 16
CHUNK = 128
DEG_W = 16


def _fill2d(ref, rows, cols, value):
    val = jnp.full((1, LANES), value, jnp.float32)

    @pl.loop(0, rows)
    def _(i):
        @pl.loop(0, cols, step=LANES)
        def _(k):
            ref.at[pl.ds(i, 1), pl.ds(k, LANES)][...] = val


def _zero_shared_slice(zbuf, shared, base, rows):
    zrows = zbuf.shape[0]

    @pl.loop(0, rows, step=zrows)
    def _(r):
        pltpu.sync_copy(zbuf, shared.at[pl.ds(base + r, zrows)])


def _deg_kernel_body(row_hbm, deg_hbm, idx_v, ones_v, zbuf, deg_sh, n_pad,
                     cpt):
    c = lax.axis_index("c")
    s = lax.axis_index("s")
    rpt = n_pad // NS
    base = s * rpt
    _fill2d(ones_v, CHUNK, DEG_W, 1.0)
    _fill2d(zbuf, zbuf.shape[0], DEG_W, 0.0)
    _zero_shared_slice(zbuf, deg_sh, base, rpt)
    pltpu.sync_copy(row_hbm.at[c, s], idx_v)
    plsc.subcore_barrier()

    @pl.loop(0, cpt)
    def _(j):
        pltpu.sync_copy(ones_v, deg_sh.at[idx_v.at[j]], add=True)

    plsc.subcore_barrier()
    pltpu.sync_copy(deg_sh.at[pl.ds(base, rpt)],
                    deg_hbm.at[c, pl.ds(base, rpt)])


NBUF = 2
IBLK = 16


def _agg_kernel_body(row_hbm, col_hbm, y_hbm, acc_hbm, idx_r, idx_c, rows_v,
                     zbuf, acc_sh, sem, n_pad, d, cpt):
    c = lax.axis_index("c")
    s = lax.axis_index("s")
    rpt = n_pad // NS
    base = s * rpt
    _fill2d(zbuf, zbuf.shape[0], d, 0.0)
    _zero_shared_slice(zbuf, acc_sh, base, rpt)
    pltpu.sync_copy(row_hbm.at[c, s], idx_r)
    pltpu.sync_copy(col_hbm.at[c, s], idx_c)
    plsc.subcore_barrier()

    @pl.loop(0, cpt)
    def _(j):
        pltpu.async_copy(y_hbm.at[idx_r.at[j]], rows_v, sem).wait()
        pltpu.sync_copy(rows_v, acc_sh.at[idx_c.at[j]], add=True)

    plsc.subcore_barrier()
    pltpu.sync_copy(acc_sh.at[pl.ds(base, rpt)],
                    acc_hbm.at[c, pl.ds(base, rpt)])


def _matmul_scale_body(x_ref, w_ref, d0_ref, d1_ref, y_ref):
    deg = d0_ref[:, 0:1] + d1_ref[:, 0:1] + 1.0
    xw = jnp.dot(x_ref[...], w_ref[...], preferred_element_type=jnp.float32)
    y_ref[...] = xw * lax.rsqrt(deg)


def _epilogue_body(a0_ref, a1_ref, y_ref, d0_ref, d1_ref, b_ref, o_ref):
    deg = d0_ref[:, 0:1] + d1_ref[:, 0:1] + 1.0
    dinv = lax.rsqrt(deg)
    o_ref[...] = dinv * (a0_ref[...] + a1_ref[...] + y_ref[...]) + b_ref[...]


def kernel(x, edge_index, W, b):
    n, d_in = x.shape
    d = W.shape[1]
    e = edge_index.shape[1]

    n_pad = ((n + 1 + 1023) // 1024) * 1024
    cpt = -(-e // (NW * CHUNK))
    cpt = ((cpt + IBLK - 1) // IBLK) * IBLK
    e_pad = NW * cpt * CHUNK

    ei = edge_index.astype(jnp.int32)
    row = jnp.concatenate([ei[0], jnp.full((e_pad - e,), n, jnp.int32)])
    col = jnp.concatenate([ei[1], jnp.full((e_pad - e,), n, jnp.int32)])
    row_t = row.reshape(NC, NS, cpt, CHUNK)
    col_t = col.reshape(NC, NS, cpt, CHUNK)
    x_pad = jnp.pad(x, ((0, n_pad - n), (0, 0)))

    mesh = plsc.VectorSubcoreMesh(core_axis_name="c", subcore_axis_name="s")

    deg_fn = pl.kernel(
        functools.partial(_deg_kernel_body, n_pad=n_pad, cpt=cpt),
        out_type=jax.ShapeDtypeStruct((NC, n_pad, DEG_W), jnp.float32),
        mesh=mesh,
        scratch_types=[
            pltpu.VMEM((cpt, CHUNK), jnp.int32),
            pltpu.VMEM((CHUNK, DEG_W), jnp.float32),
            pltpu.VMEM((64, DEG_W), jnp.float32),
            pltpu.VMEM_SHARED((n_pad, DEG_W), jnp.float32),
        ],
    )
    deg = deg_fn(row_t)

    bm = 512
    grid = (n_pad // bm,)
    y = pl.pallas_call(
        _matmul_scale_body,
        grid=grid,
        in_specs=[
            pl.BlockSpec((bm, d_in), lambda i: (i, 0)),
            pl.BlockSpec((d_in, d), lambda i: (0, 0)),
            pl.BlockSpec((bm, DEG_W), lambda i: (i, 0)),
            pl.BlockSpec((bm, DEG_W), lambda i: (i, 0)),
        ],
        out_specs=pl.BlockSpec((bm, d), lambda i: (i, 0)),
        out_shape=jax.ShapeDtypeStruct((n_pad, d), jnp.float32),
    )(x_pad, W, deg[0], deg[1])

    agg_fn = pl.kernel(
        functools.partial(_agg_kernel_body, n_pad=n_pad, d=d, cpt=cpt),
        out_type=jax.ShapeDtypeStruct((NC, n_pad, d), jnp.float32),
        mesh=mesh,
        scratch_types=[
            pltpu.VMEM((cpt, CHUNK), jnp.int32),
            pltpu.VMEM((cpt, CHUNK), jnp.int32),
            pltpu.VMEM((CHUNK, d), jnp.float32),
            pltpu.VMEM((32, d), jnp.float32),
            pltpu.VMEM_SHARED((n_pad, d), jnp.float32),
            pltpu.SemaphoreType.DMA,
        ],
    )
    acc = agg_fn(row_t, col_t, y)

    out = pl.pallas_call(
        _epilogue_body,
        grid=grid,
        in_specs=[
            pl.BlockSpec((bm, d), lambda i: (i, 0)),
            pl.BlockSpec((bm, d), lambda i: (i, 0)),
            pl.BlockSpec((bm, d), lambda i: (i, 0)),
            pl.BlockSpec((bm, DEG_W), lambda i: (i, 0)),
            pl.BlockSpec((bm, DEG_W), lambda i: (i, 0)),
            pl.BlockSpec((1, d), lambda i: (0, 0)),
        ],
        out_specs=pl.BlockSpec((bm, d), lambda i: (i, 0)),
        out_shape=jax.ShapeDtypeStruct((n_pad, d), jnp.float32),
    )(acc[0], acc[1], y, deg[0], deg[1], b.reshape(1, d))

    return out[:n]


# speedup vs baseline: 2.1668x; 2.1668x over previous
import functools

import jax
import jax.numpy as jnp
from jax import lax
from jax.experimental import pallas as pl
from jax.experimental.pallas import tpu as pltpu
from jax.experimental.pallas import tpu_sc as plsc

NC = 2
NS = 16
NW = NC * NS
LANES = 16
CHUNK = 128
DEG_W = 16


def _fill2d(ref, rows, cols, value):
    val = jnp.full((1, LANES), value, jnp.float32)

    @pl.loop(0, rows)
    def _(i):
        @pl.loop(0, cols, step=LANES)
        def _(k):
            ref.at[pl.ds(i, 1), pl.ds(k, LANES)][...] = val


def _zero_shared_slice(zbuf, shared, base, rows):
    zrows = zbuf.shape[0]

    @pl.loop(0, rows, step=zrows)
    def _(r):
        pltpu.sync_copy(zbuf, shared.at[pl.ds(base + r, zrows)])


def _deg_kernel_body(row_hbm, deg_hbm, idx_v, ones_v, zbuf, deg_sh, n_pad,
                     cpt):
    c = lax.axis_index("c")
    s = lax.axis_index("s")
    rpt = n_pad // NS
    base = s * rpt
    _fill2d(ones_v, CHUNK, DEG_W, 1.0)
    _fill2d(zbuf, zbuf.shape[0], DEG_W, 0.0)
    _zero_shared_slice(zbuf, deg_sh, base, rpt)
    pltpu.sync_copy(row_hbm.at[c, s], idx_v)
    plsc.subcore_barrier()

    @pl.loop(0, cpt)
    def _(j):
        pltpu.sync_copy(ones_v, deg_sh.at[idx_v.at[j]], add=True)

    plsc.subcore_barrier()
    pltpu.sync_copy(deg_sh.at[pl.ds(base, rpt)],
                    deg_hbm.at[c, pl.ds(base, rpt)])


def _agg_kernel_body(row_hbm, col_hbm, y_hbm, acc_hbm, idx_r, idx_c, rows_v,
                     zbuf, acc_sh, sem, n_pad, d, cpt):
    c = lax.axis_index("c")
    s = lax.axis_index("s")
    rpt = n_pad // NS
    base = s * rpt
    _fill2d(zbuf, zbuf.shape[0], d, 0.0)
    _zero_shared_slice(zbuf, acc_sh, base, rpt)
    pltpu.sync_copy(row_hbm.at[c, s], idx_r)
    pltpu.sync_copy(col_hbm.at[c, s], idx_c)
    plsc.subcore_barrier()

    @pl.loop(0, cpt)
    def _(j):
        pltpu.async_copy(y_hbm.at[idx_r.at[j]], rows_v, sem).wait()
        pltpu.sync_copy(rows_v, acc_sh.at[idx_c.at[j]], add=True)

    plsc.subcore_barrier()
    pltpu.sync_copy(acc_sh.at[pl.ds(base, rpt)],
                    acc_hbm.at[c, pl.ds(base, rpt)])


def _matmul_body(x_ref, w_ref, o_ref):
    o_ref[...] = jnp.dot(x_ref[...], w_ref[...],
                         preferred_element_type=jnp.float32)


def _scale_body(xw_ref, d0_ref, d1_ref, y_ref):
    deg = d0_ref[:, 0:1] + d1_ref[:, 0:1] + 1.0
    y_ref[...] = xw_ref[...] * lax.rsqrt(deg)


def _epilogue_body(a0_ref, a1_ref, y_ref, d0_ref, d1_ref, b_ref, o_ref):
    deg = d0_ref[:, 0:1] + d1_ref[:, 0:1] + 1.0
    dinv = lax.rsqrt(deg)
    o_ref[...] = dinv * (a0_ref[...] + a1_ref[...] + y_ref[...]) + b_ref[...]


def kernel(x, edge_index, W, b):
    n, d_in = x.shape
    d = W.shape[1]
    e = edge_index.shape[1]

    n_pad = ((n + 1 + 1023) // 1024) * 1024
    cpt = -(-e // (NW * CHUNK))
    e_pad = NW * cpt * CHUNK

    ei = edge_index.astype(jnp.int32)
    pad_tgt = n + jnp.arange(e_pad - e, dtype=jnp.int32) % (n_pad - n)
    row = jnp.concatenate([ei[0], pad_tgt])
    col = jnp.concatenate([ei[1], pad_tgt])
    row_t = row.reshape(cpt, NC, NS, CHUNK).transpose(1, 2, 0, 3)
    col_t = col.reshape(cpt, NC, NS, CHUNK).transpose(1, 2, 0, 3)
    x_pad = jnp.pad(x, ((0, n_pad - n), (0, 0)))

    mesh = plsc.VectorSubcoreMesh(core_axis_name="c", subcore_axis_name="s")

    deg_fn = pl.kernel(
        functools.partial(_deg_kernel_body, n_pad=n_pad, cpt=cpt),
        out_type=jax.ShapeDtypeStruct((NC, n_pad, DEG_W), jnp.float32),
        mesh=mesh,
        scratch_types=[
            pltpu.VMEM((cpt, CHUNK), jnp.int32),
            pltpu.VMEM((CHUNK, DEG_W), jnp.float32),
            pltpu.VMEM((64, DEG_W), jnp.float32),
            pltpu.VMEM_SHARED((n_pad, DEG_W), jnp.float32),
        ],
    )
    deg = deg_fn(row_t)

    bm = 512
    grid = (n_pad // bm,)
    xw = pl.pallas_call(
        _matmul_body,
        grid=grid,
        in_specs=[
            pl.BlockSpec((bm, d_in), lambda i: (i, 0)),
            pl.BlockSpec((d_in, d), lambda i: (0, 0)),
        ],
        out_specs=pl.BlockSpec((bm, d), lambda i: (i, 0)),
        out_shape=jax.ShapeDtypeStruct((n_pad, d), jnp.float32),
    )(x_pad, W)

    y = pl.pallas_call(
        _scale_body,
        grid=grid,
        in_specs=[
            pl.BlockSpec((bm, d), lambda i: (i, 0)),
            pl.BlockSpec((bm, DEG_W), lambda i: (i, 0)),
            pl.BlockSpec((bm, DEG_W), lambda i: (i, 0)),
        ],
        out_specs=pl.BlockSpec((bm, d), lambda i: (i, 0)),
        out_shape=jax.ShapeDtypeStruct((n_pad, d), jnp.float32),
    )(xw, deg[0], deg[1])

    agg_fn = pl.kernel(
        functools.partial(_agg_kernel_body, n_pad=n_pad, d=d, cpt=cpt),
        out_type=jax.ShapeDtypeStruct((NC, n_pad, d), jnp.float32),
        mesh=mesh,
        scratch_types=[
            pltpu.VMEM((cpt, CHUNK), jnp.int32),
            pltpu.VMEM((cpt, CHUNK), jnp.int32),
            pltpu.VMEM((CHUNK, d), jnp.float32),
            pltpu.VMEM((64, d), jnp.float32),
            pltpu.VMEM_SHARED((n_pad, d), jnp.float32),
            pltpu.SemaphoreType.DMA,
        ],
    )
    acc = agg_fn(row_t, col_t, y)

    out = pl.pallas_call(
        _epilogue_body,
        grid=grid,
        in_specs=[
            pl.BlockSpec((bm, d), lambda i: (i, 0)),
            pl.BlockSpec((bm, d), lambda i: (i, 0)),
            pl.BlockSpec((bm, d), lambda i: (i, 0)),
            pl.BlockSpec((bm, DEG_W), lambda i: (i, 0)),
            pl.BlockSpec((bm, DEG_W), lambda i: (i, 0)),
            pl.BlockSpec((1, d), lambda i: (0, 0)),
        ],
        out_specs=pl.BlockSpec((bm, d), lambda i: (i, 0)),
        out_shape=jax.ShapeDtypeStruct((n_pad, d), jnp.float32),
    )(acc[0], acc[1], y, deg[0], deg[1], b.reshape(1, d))

    return out[:n]


# speedup vs baseline: 2.7999x; 1.2922x over previous
import functools

import jax
import jax.numpy as jnp
from jax import lax
from jax.experimental import pallas as pl
from jax.experimental.pallas import tpu as pltpu
from jax.experimental.pallas import tpu_sc as plsc

NC = 2
NS = 16
NW = NC * NS
LANES = 16
CHUNK = 128
DEG_W = 16


def _fill2d(ref, rows, cols, value):
    val = jnp.full((1, LANES), value, jnp.float32)

    @pl.loop(0, rows)
    def _(i):
        @pl.loop(0, cols, step=LANES)
        def _(k):
            ref.at[pl.ds(i, 1), pl.ds(k, LANES)][...] = val


def _zero_shared_slice(zbuf, shared, base, rows):
    zrows = zbuf.shape[0]

    @pl.loop(0, rows, step=zrows)
    def _(r):
        pltpu.sync_copy(zbuf, shared.at[pl.ds(base + r, zrows)])


def _deg_kernel_body(row_hbm, deg_hbm, idx_v, ones_v, zbuf, deg_sh, n_pad,
                     cpt):
    c = lax.axis_index("c")
    s = lax.axis_index("s")
    rpt = n_pad // NS
    base = s * rpt
    _fill2d(ones_v, CHUNK, DEG_W, 1.0)
    _fill2d(zbuf, zbuf.shape[0], DEG_W, 0.0)
    _zero_shared_slice(zbuf, deg_sh, base, rpt)
    pltpu.sync_copy(row_hbm.at[c, s], idx_v)
    plsc.subcore_barrier()

    @pl.loop(0, cpt)
    def _(j):
        pltpu.sync_copy(ones_v, deg_sh.at[idx_v.at[j]], add=True)

    plsc.subcore_barrier()
    pltpu.sync_copy(deg_sh.at[pl.ds(base, rpt)],
                    deg_hbm.at[c, pl.ds(base, rpt)])


def _agg_kernel_body(row_hbm, col_hbm, y_hbm, acc_hbm, idx_r, idx_c, buf0,
                     buf1, zbuf, acc_sh, sem0, sem1, n_pad, d, cpt):
    c = lax.axis_index("c")
    s = lax.axis_index("s")
    rpt = n_pad // NS
    base = s * rpt
    ph = cpt // 2
    _fill2d(zbuf, zbuf.shape[0], d, 0.0)
    _zero_shared_slice(zbuf, acc_sh, base, rpt)
    plsc.subcore_barrier()

    for p in range(2):
        off = p * ph
        pltpu.sync_copy(row_hbm.at[c, s, pl.ds(off, ph)], idx_r)
        pltpu.sync_copy(col_hbm.at[c, s, pl.ds(off, ph)], idx_c)
        pltpu.async_copy(y_hbm.at[idx_r.at[0]], buf0, sem0)
        pltpu.async_copy(y_hbm.at[idx_r.at[1]], buf1, sem1)

        @pl.loop(0, ph - 2, step=2)
        def _(j):
            pltpu.make_async_copy(y_hbm.at[idx_r.at[j]], buf0, sem0).wait()
            pltpu.sync_copy(buf0, acc_sh.at[idx_c.at[j]], add=True)
            pltpu.async_copy(y_hbm.at[idx_r.at[j + 2]], buf0, sem0)
            pltpu.make_async_copy(y_hbm.at[idx_r.at[j + 1]], buf1,
                                  sem1).wait()
            pltpu.sync_copy(buf1, acc_sh.at[idx_c.at[j + 1]], add=True)
            pltpu.async_copy(y_hbm.at[idx_r.at[j + 3]], buf1, sem1)

        pltpu.make_async_copy(y_hbm.at[idx_r.at[ph - 2]], buf0, sem0).wait()
        pltpu.sync_copy(buf0, acc_sh.at[idx_c.at[ph - 2]], add=True)
        pltpu.make_async_copy(y_hbm.at[idx_r.at[ph - 1]], buf1, sem1).wait()
        pltpu.sync_copy(buf1, acc_sh.at[idx_c.at[ph - 1]], add=True)

    plsc.subcore_barrier()
    pltpu.sync_copy(acc_sh.at[pl.ds(base, rpt)],
                    acc_hbm.at[c, pl.ds(base, rpt)])


def _matmul_scale_body(x_ref, w_ref, d0_ref, d1_ref, y_ref):
    deg = d0_ref[:, 0:1] + d1_ref[:, 0:1] + 1.0
    xw = jnp.dot(x_ref[...], w_ref[...], preferred_element_type=jnp.float32)
    y_ref[...] = xw * lax.rsqrt(deg)


def _epilogue_body(a0_ref, a1_ref, y_ref, d0_ref, d1_ref, b_ref, o_ref):
    deg = d0_ref[:, 0:1] + d1_ref[:, 0:1] + 1.0
    dinv = lax.rsqrt(deg)
    o_ref[...] = dinv * (a0_ref[...] + a1_ref[...] + y_ref[...]) + b_ref[...]


def kernel(x, edge_index, W, b):
    n, d_in = x.shape
    d = W.shape[1]
    e = edge_index.shape[1]

    n_pad = ((n + 1 + 1023) // 1024) * 1024
    cpt = -(-e // (NW * CHUNK))
    cpt += cpt % 2
    e_pad = NW * cpt * CHUNK

    ei = edge_index.astype(jnp.int32)
    pad_tgt = n + jnp.arange(e_pad - e, dtype=jnp.int32) % (n_pad - n)
    row = jnp.concatenate([ei[0], pad_tgt])
    col = jnp.concatenate([ei[1], pad_tgt])
    row_t = row.reshape(cpt, NC, NS, CHUNK).transpose(1, 2, 0, 3)
    col_t = col.reshape(cpt, NC, NS, CHUNK).transpose(1, 2, 0, 3)
    x_pad = jnp.pad(x, ((0, n_pad - n), (0, 0)))

    mesh = plsc.VectorSubcoreMesh(core_axis_name="c", subcore_axis_name="s")

    deg_fn = pl.kernel(
        functools.partial(_deg_kernel_body, n_pad=n_pad, cpt=cpt),
        out_type=jax.ShapeDtypeStruct((NC, n_pad, DEG_W), jnp.float32),
        mesh=mesh,
        scratch_types=[
            pltpu.VMEM((cpt, CHUNK), jnp.int32),
            pltpu.VMEM((CHUNK, DEG_W), jnp.float32),
            pltpu.VMEM((64, DEG_W), jnp.float32),
            pltpu.VMEM_SHARED((n_pad, DEG_W), jnp.float32),
        ],
    )
    deg = deg_fn(row_t)

    bm = 512
    grid = (n_pad // bm,)
    y = pl.pallas_call(
        _matmul_scale_body,
        grid=grid,
        in_specs=[
            pl.BlockSpec((bm, d_in), lambda i: (i, 0)),
            pl.BlockSpec((d_in, d), lambda i: (0, 0)),
            pl.BlockSpec((bm, DEG_W), lambda i: (i, 0)),
            pl.BlockSpec((bm, DEG_W), lambda i: (i, 0)),
        ],
        out_specs=pl.BlockSpec((bm, d), lambda i: (i, 0)),
        out_shape=jax.ShapeDtypeStruct((n_pad, d), jnp.float32),
    )(x_pad, W, deg[0], deg[1])

    agg_fn = pl.kernel(
        functools.partial(_agg_kernel_body, n_pad=n_pad, d=d, cpt=cpt),
        out_type=jax.ShapeDtypeStruct((NC, n_pad, d), jnp.float32),
        mesh=mesh,
        scratch_types=[
            pltpu.VMEM((cpt // 2, CHUNK), jnp.int32),
            pltpu.VMEM((cpt // 2, CHUNK), jnp.int32),
            pltpu.VMEM((CHUNK, d), jnp.float32),
            pltpu.VMEM((CHUNK, d), jnp.float32),
            pltpu.VMEM((32, d), jnp.float32),
            pltpu.VMEM_SHARED((n_pad, d), jnp.float32),
            pltpu.SemaphoreType.DMA,
            pltpu.SemaphoreType.DMA,
        ],
    )
    acc = agg_fn(row_t, col_t, y)

    out = pl.pallas_call(
        _epilogue_body,
        grid=grid,
        in_specs=[
            pl.BlockSpec((bm, d), lambda i: (i, 0)),
            pl.BlockSpec((bm, d), lambda i: (i, 0)),
            pl.BlockSpec((bm, d), lambda i: (i, 0)),
            pl.BlockSpec((bm, DEG_W), lambda i: (i, 0)),
            pl.BlockSpec((bm, DEG_W), lambda i: (i, 0)),
            pl.BlockSpec((1, d), lambda i: (0, 0)),
        ],
        out_specs=pl.BlockSpec((bm, d), lambda i: (i, 0)),
        out_shape=jax.ShapeDtypeStruct((n_pad, d), jnp.float32),
    )(acc[0], acc[1], y, deg[0], deg[1], b.reshape(1, d))

    return out[:n]


# speedup vs baseline: 2.8516x; 1.0185x over previous
import functools

import jax
import jax.numpy as jnp
from jax import lax
from jax.experimental import pallas as pl
from jax.experimental.pallas import tpu as pltpu
from jax.experimental.pallas import tpu_sc as plsc

NC = 2
NS = 16
NW = NC * NS
LANES = 16
CHUNK = 128
DEG_W = 16


def _fill2d(ref, rows, cols, value):
    val = jnp.full((1, LANES), value, jnp.float32)

    @pl.loop(0, rows)
    def _(i):
        @pl.loop(0, cols, step=LANES)
        def _(k):
            ref.at[pl.ds(i, 1), pl.ds(k, LANES)][...] = val


def _zero_shared_slice(zbuf, shared, base, rows):
    zrows = zbuf.shape[0]

    @pl.loop(0, rows, step=zrows)
    def _(r):
        pltpu.sync_copy(zbuf, shared.at[pl.ds(base + r, zrows)])


def _deg_kernel_body(row_hbm, deg_hbm, idx_v, ones_v, zbuf, deg_sh, sem,
                     n_pad, cpt):
    c = lax.axis_index("c")
    s = lax.axis_index("s")
    rpt = n_pad // NS
    base = s * rpt
    _fill2d(ones_v, CHUNK, DEG_W, 1.0)
    _fill2d(zbuf, zbuf.shape[0], DEG_W, 0.0)
    _zero_shared_slice(zbuf, deg_sh, base, rpt)
    pltpu.sync_copy(row_hbm.at[c, s], idx_v)
    plsc.subcore_barrier()

    @pl.loop(0, cpt)
    def _(j):
        pltpu.async_copy(ones_v, deg_sh.at[idx_v.at[j]], sem, add=True)

    @pl.loop(0, cpt)
    def _(j):
        pltpu.make_async_copy(ones_v, deg_sh.at[idx_v.at[j]], sem).wait()

    plsc.subcore_barrier()
    pltpu.sync_copy(deg_sh.at[pl.ds(base, rpt)],
                    deg_hbm.at[c, pl.ds(base, rpt)])


def _agg_kernel_body(row_hbm, col_hbm, y_hbm, acc_hbm, idx_r, idx_c, buf0,
                     buf1, zbuf, acc_sh, sem0, sem1, n_pad, d, cpt):
    c = lax.axis_index("c")
    s = lax.axis_index("s")
    rpt = n_pad // NS
    base = s * rpt
    ph = cpt // 2
    _fill2d(zbuf, zbuf.shape[0], d, 0.0)
    _zero_shared_slice(zbuf, acc_sh, base, rpt)
    plsc.subcore_barrier()

    for p in range(2):
        off = p * ph
        pltpu.sync_copy(row_hbm.at[c, s, pl.ds(off, ph)], idx_r)
        pltpu.sync_copy(col_hbm.at[c, s, pl.ds(off, ph)], idx_c)
        pltpu.async_copy(y_hbm.at[idx_r.at[0]], buf0, sem0)
        pltpu.async_copy(y_hbm.at[idx_r.at[1]], buf1, sem1)

        @pl.loop(0, ph - 2, step=2)
        def _(j):
            pltpu.make_async_copy(y_hbm.at[idx_r.at[j]], buf0, sem0).wait()
            pltpu.sync_copy(buf0, acc_sh.at[idx_c.at[j]], add=True)
            pltpu.async_copy(y_hbm.at[idx_r.at[j + 2]], buf0, sem0)
            pltpu.make_async_copy(y_hbm.at[idx_r.at[j + 1]], buf1,
                                  sem1).wait()
            pltpu.sync_copy(buf1, acc_sh.at[idx_c.at[j + 1]], add=True)
            pltpu.async_copy(y_hbm.at[idx_r.at[j + 3]], buf1, sem1)

        pltpu.make_async_copy(y_hbm.at[idx_r.at[ph - 2]], buf0, sem0).wait()
        pltpu.sync_copy(buf0, acc_sh.at[idx_c.at[ph - 2]], add=True)
        pltpu.make_async_copy(y_hbm.at[idx_r.at[ph - 1]], buf1, sem1).wait()
        pltpu.sync_copy(buf1, acc_sh.at[idx_c.at[ph - 1]], add=True)

    plsc.subcore_barrier()
    pltpu.sync_copy(acc_sh.at[pl.ds(base, rpt)],
                    acc_hbm.at[c, pl.ds(base, rpt)])


def _matmul_scale_body(x_ref, w_ref, d0_ref, d1_ref, y_ref):
    deg = d0_ref[:, 0:1] + d1_ref[:, 0:1] + 1.0
    xw = jnp.dot(x_ref[...], w_ref[...], preferred_element_type=jnp.float32)
    y_ref[...] = xw * lax.rsqrt(deg)


def _epilogue_body(a0_ref, a1_ref, y_ref, d0_ref, d1_ref, b_ref, o_ref):
    deg = d0_ref[:, 0:1] + d1_ref[:, 0:1] + 1.0
    dinv = lax.rsqrt(deg)
    o_ref[...] = dinv * (a0_ref[...] + a1_ref[...] + y_ref[...]) + b_ref[...]


def kernel(x, edge_index, W, b):
    n, d_in = x.shape
    d = W.shape[1]
    e = edge_index.shape[1]

    n_pad = ((n + 1 + 1023) // 1024) * 1024
    cpt = -(-e // (NW * CHUNK))
    cpt += cpt % 2
    e_pad = NW * cpt * CHUNK

    ei = edge_index.astype(jnp.int32)
    pad_tgt = n + jnp.arange(e_pad - e, dtype=jnp.int32) % (n_pad - n)
    row = jnp.concatenate([ei[0], pad_tgt])
    col = jnp.concatenate([ei[1], pad_tgt])
    row_t = row.reshape(cpt, NC, NS, CHUNK).transpose(1, 2, 0, 3)
    col_t = col.reshape(cpt, NC, NS, CHUNK).transpose(1, 2, 0, 3)
    x_pad = jnp.pad(x, ((0, n_pad - n), (0, 0)))

    mesh = plsc.VectorSubcoreMesh(core_axis_name="c", subcore_axis_name="s")

    deg_fn = pl.kernel(
        functools.partial(_deg_kernel_body, n_pad=n_pad, cpt=cpt),
        out_type=jax.ShapeDtypeStruct((NC, n_pad, DEG_W), jnp.float32),
        mesh=mesh,
        scratch_types=[
            pltpu.VMEM((cpt, CHUNK), jnp.int32),
            pltpu.VMEM((CHUNK, DEG_W), jnp.float32),
            pltpu.VMEM((64, DEG_W), jnp.float32),
            pltpu.VMEM_SHARED((n_pad, DEG_W), jnp.float32),
            pltpu.SemaphoreType.DMA,
        ],
    )
    deg = deg_fn(row_t)

    bm = 512
    grid = (n_pad // bm,)
    y = pl.pallas_call(
        _matmul_scale_body,
        grid=grid,
        in_specs=[
            pl.BlockSpec((bm, d_in), lambda i: (i, 0)),
            pl.BlockSpec((d_in, d), lambda i: (0, 0)),
            pl.BlockSpec((bm, DEG_W), lambda i: (i, 0)),
            pl.BlockSpec((bm, DEG_W), lambda i: (i, 0)),
        ],
        out_specs=pl.BlockSpec((bm, d), lambda i: (i, 0)),
        out_shape=jax.ShapeDtypeStruct((n_pad, d), jnp.float32),
    )(x_pad, W, deg[0], deg[1])

    agg_fn = pl.kernel(
        functools.partial(_agg_kernel_body, n_pad=n_pad, d=d, cpt=cpt),
        out_type=jax.ShapeDtypeStruct((NC, n_pad, d), jnp.float32),
        mesh=mesh,
        scratch_types=[
            pltpu.VMEM((cpt // 2, CHUNK), jnp.int32),
            pltpu.VMEM((cpt // 2, CHUNK), jnp.int32),
            pltpu.VMEM((CHUNK, d), jnp.float32),
            pltpu.VMEM((CHUNK, d), jnp.float32),
            pltpu.VMEM((32, d), jnp.float32),
            pltpu.VMEM_SHARED((n_pad, d), jnp.float32),
            pltpu.SemaphoreType.DMA,
            pltpu.SemaphoreType.DMA,
        ],
    )
    acc = agg_fn(row_t, col_t, y)

    out = pl.pallas_call(
        _epilogue_body,
        grid=grid,
        in_specs=[
            pl.BlockSpec((bm, d), lambda i: (i, 0)),
            pl.BlockSpec((bm, d), lambda i: (i, 0)),
            pl.BlockSpec((bm, d), lambda i: (i, 0)),
            pl.BlockSpec((bm, DEG_W), lambda i: (i, 0)),
            pl.BlockSpec((bm, DEG_W), lambda i: (i, 0)),
            pl.BlockSpec((1, d), lambda i: (0, 0)),
        ],
        out_specs=pl.BlockSpec((bm, d), lambda i: (i, 0)),
        out_shape=jax.ShapeDtypeStruct((n_pad, d), jnp.float32),
    )(acc[0], acc[1], y, deg[0], deg[1], b.reshape(1, d))

    return out[:n]
